# baseline (speedup 1.0000x reference)
import functools

import jax
import jax.numpy as jnp
from jax import lax
from jax.experimental import pallas as pl
from jax.experimental.pallas import tpu as pltpu
from jax.experimental.pallas import tpu_sc as plsc

N = 10000
D = 128
CH = 128
NC = 2
NS = 16
NW = NC * NS
ACC = 10240
DUMMY = N
ZR = ACC // NS
NZB = ZR // CH
BLK = 1000


def _sc_scatter_body(y_h, src_h, dst_h, out_h, src_v, dst_v, bufa, bufb,
                     acc, sema, semb):
    c = lax.axis_index("c")
    s = lax.axis_index("s")
    w = c * NS + s
    nch2 = src_v.shape[0]

    z16 = jnp.zeros((16,), jnp.float32)

    def zrow(i, carry):
        for k in range(D // 16):
            bufa[i, pl.ds(k * 16, 16)] = z16
        return carry

    lax.fori_loop(0, CH, zrow, 0)
    base = s * ZR

    def zblk(k, carry):
        pltpu.sync_copy(bufa, acc.at[pl.ds(base + k * CH, CH)])
        return carry

    lax.fori_loop(0, NZB, zblk, 0)
    plsc.subcore_barrier()

    for h in range(2):
        pltpu.sync_copy(src_h.at[w, pl.ds(h * nch2, nch2)], src_v)
        pltpu.sync_copy(dst_h.at[w, pl.ds(h * nch2, nch2)], dst_v)

        def step(j, carry):
            pltpu.async_copy(y_h.at[src_v.at[j]], bufa, sema).wait()
            pltpu.sync_copy(bufa, acc.at[dst_v.at[j]], add=True)
            return carry

        lax.fori_loop(0, nch2, step, 0)
    plsc.subcore_barrier()

    def dblk(k, carry):
        pltpu.sync_copy(acc.at[pl.ds(base + k * CH, CH)], bufa)
        pltpu.sync_copy(bufa, out_h.at[c].at[pl.ds(base + k * CH, CH)])
        return carry

    lax.fori_loop(0, NZB, dblk, 0)


def _sc_deg_body(dst_h, out_h, dst_v, ones_v, zb, acc):
    c = lax.axis_index("c")
    s = lax.axis_index("s")
    w = c * NS + s
    nch = dst_v.shape[0]
    pltpu.sync_copy(dst_h.at[w], dst_v)

    one16 = jnp.ones((16,), jnp.float32)
    for k in range(CH // 16):
        ones_v[pl.ds(k * 16, 16)] = one16
    z16 = jnp.zeros((16,), jnp.float32)
    for k in range(ZR // 16):
        zb[pl.ds(k * 16, 16)] = z16
    base = s * ZR
    pltpu.sync_copy(zb, acc.at[pl.ds(base, ZR)])
    plsc.subcore_barrier()

    def step(j, carry):
        pltpu.sync_copy(ones_v, acc.at[dst_v.at[j]], add=True)
        return carry

    lax.fori_loop(0, nch, step, 0)
    plsc.subcore_barrier()

    pltpu.sync_copy(acc.at[pl.ds(base, ZR)], zb)
    pltpu.sync_copy(zb, out_h.at[c].at[pl.ds(base, ZR)])


@functools.cache
def _mesh():
    return plsc.VectorSubcoreMesh(core_axis_name="c", subcore_axis_name="s",
                                  num_cores=NC, num_subcores=NS)


def _sc_scatter(y, src_g, dst_g):
    nch2 = src_g.shape[1] // 2
    f = pl.kernel(
        _sc_scatter_body,
        out_type=jax.ShapeDtypeStruct((NC, ACC, D), jnp.float32),
        mesh=_mesh(),
        scratch_types=[
            pltpu.VMEM((nch2, CH), jnp.int32),
            pltpu.VMEM((nch2, CH), jnp.int32),
            pltpu.VMEM((CH, D), jnp.float32),
            pltpu.VMEM((CH, D), jnp.float32),
            pltpu.VMEM_SHARED((ACC, D), jnp.float32),
            pltpu.SemaphoreType.DMA,
            pltpu.SemaphoreType.DMA,
        ],
    )
    return f(y, src_g, dst_g)


def _sc_deg(dst_g):
    nch = dst_g.shape[1]
    f = pl.kernel(
        _sc_deg_body,
        out_type=jax.ShapeDtypeStruct((NC, ACC), jnp.float32),
        mesh=_mesh(),
        scratch_types=[
            pltpu.VMEM((nch, CH), jnp.int32),
            pltpu.VMEM((CH,), jnp.float32),
            pltpu.VMEM((ZR,), jnp.float32),
            pltpu.VMEM_SHARED((ACC,), jnp.float32),
        ],
    )
    return f(dst_g)


def _tc_prep_body(deg_ref, x_ref, w_ref, y_ref, dinv_ref):
    deg = deg_ref[0, :, :] + deg_ref[1, :, :]
    dinv = lax.rsqrt(jnp.maximum(deg, 1.0))
    y_ref[...] = jnp.dot(x_ref[...] * dinv, w_ref[...],
                         preferred_element_type=jnp.float32)
    dinv_ref[...] = dinv


def _tc_mid_body(s_ref, dinv_ref, b_ref, w_ref, y_ref):
    dinv = dinv_ref[...]
    a = (s_ref[0, :, :] + s_ref[1, :, :]) * dinv + b_ref[...]
    mu = jnp.mean(a, axis=1, keepdims=True)
    d = a - mu
    var = jnp.mean(d * d, axis=1, keepdims=True)
    h = d * lax.rsqrt(var + 1e-5)
    h = jnp.maximum(h, 0.0)
    y_ref[...] = jnp.dot(h * dinv, w_ref[...],
                         preferred_element_type=jnp.float32)


def _tc_last_body(s_ref, dinv_ref, b_ref, o_ref):
    a = (s_ref[0, :, :] + s_ref[1, :, :]) * dinv_ref[...] + b_ref[...]
    mu = jnp.mean(a, axis=1, keepdims=True)
    d = a - mu
    var = jnp.mean(d * d, axis=1, keepdims=True)
    o_ref[...] = d * lax.rsqrt(var + 1e-5)


def _tc_prep(deg2, feat, W0):
    return pl.pallas_call(
        _tc_prep_body,
        grid=(N // BLK,),
        in_specs=[
            pl.BlockSpec((2, BLK, 1), lambda i: (0, i, 0)),
            pl.BlockSpec((BLK, D), lambda i: (i, 0)),
            pl.BlockSpec((D, D), lambda i: (0, 0)),
        ],
        out_specs=[
            pl.BlockSpec((BLK, D), lambda i: (i, 0)),
            pl.BlockSpec((BLK, 1), lambda i: (i, 0)),
        ],
        out_shape=[
            jax.ShapeDtypeStruct((N, D), jnp.float32),
            jax.ShapeDtypeStruct((N, 1), jnp.float32),
        ],
    )(deg2, feat, W0)


def _tc_mid(s2, dinv, b, Wn):
    return pl.pallas_call(
        _tc_mid_body,
        grid=(N // BLK,),
        in_specs=[
            pl.BlockSpec((2, BLK, D), lambda i: (0, i, 0)),
            pl.BlockSpec((BLK, 1), lambda i: (i, 0)),
            pl.BlockSpec((1, D), lambda i: (0, 0)),
            pl.BlockSpec((D, D), lambda i: (0, 0)),
        ],
        out_specs=pl.BlockSpec((BLK, D), lambda i: (i, 0)),
        out_shape=jax.ShapeDtypeStruct((N, D), jnp.float32),
    )(s2, dinv, b, Wn)


def _tc_last(s2, dinv, b):
    return pl.pallas_call(
        _tc_last_body,
        grid=(N // BLK,),
        in_specs=[
            pl.BlockSpec((2, BLK, D), lambda i: (0, i, 0)),
            pl.BlockSpec((BLK, 1), lambda i: (i, 0)),
            pl.BlockSpec((1, D), lambda i: (0, 0)),
        ],
        out_specs=pl.BlockSpec((BLK, D), lambda i: (i, 0)),
        out_shape=jax.ShapeDtypeStruct((N, D), jnp.float32),
    )(s2, dinv, b)


def kernel(feat, edge_index, W0, b0, W1, b1, W2, b2):
    feat = feat.astype(jnp.float32)
    src = edge_index[0].astype(jnp.int32)
    dst = edge_index[1].astype(jnp.int32)
    e = src.shape[0]
    gran = NW * CH * 4
    epad = gran * ((e + gran - 1) // gran)
    src_g = jnp.concatenate(
        [src, jnp.zeros((epad - e,), jnp.int32)]).reshape(NW, -1, CH)
    pad_dst = DUMMY + jnp.arange(epad - e, dtype=jnp.int32) % (ACC - N)
    dst_g = jnp.concatenate([dst, pad_dst]).reshape(NW, -1, CH)

    deg2 = _sc_deg(dst_g)
    y, dinv = _tc_prep(deg2.reshape(NC, ACC, 1), feat, W0)
    s2 = _sc_scatter(y, src_g, dst_g)
    y = _tc_mid(s2, dinv, b0.reshape(1, D), W1)
    s2 = _sc_scatter(y, src_g, dst_g)
    y = _tc_mid(s2, dinv, b1.reshape(1, D), W2)
    s2 = _sc_scatter(y, src_g, dst_g)
    return _tc_last(s2, dinv, b2.reshape(1, D))


# speedup vs baseline: 1.4553x; 1.4553x over previous
import functools

import jax
import jax.numpy as jnp
from jax import lax
from jax.experimental import pallas as pl
from jax.experimental.pallas import tpu as pltpu
from jax.experimental.pallas import tpu_sc as plsc

N = 10000
D = 128
CH = 128
NC = 2
NS = 16
NW = NC * NS
ACC = 10240
DUMMY = N
ZR = ACC // NS
NZB = ZR // CH
BLK = 1000


def _sc_scatter_body(y_h, src_h, dst_h, out_h, src_v, dst_v, bufa, acc, sema):
    c = lax.axis_index("c")
    s = lax.axis_index("s")
    w = c * NS + s
    nch = src_v.shape[0]

    z16 = jnp.zeros((16,), jnp.float32)

    def zrow(i, carry):
        for k in range(D // 16):
            bufa[i, pl.ds(k * 16, 16)] = z16
        return carry

    lax.fori_loop(0, CH, zrow, 0)
    base = s * ZR

    def zblk(k, carry):
        pltpu.sync_copy(bufa, acc.at[pl.ds(base + k * CH, CH)])
        return carry

    lax.fori_loop(0, NZB, zblk, 0)
    plsc.subcore_barrier()

    pltpu.sync_copy(src_h.at[w], src_v)
    pltpu.sync_copy(dst_h.at[w], dst_v)

    def step(j, carry):
        pltpu.async_copy(y_h.at[src_v.at[j]], bufa, sema).wait()
        pltpu.sync_copy(bufa, acc.at[dst_v.at[j]], add=True)
        return carry

    lax.fori_loop(0, nch, step, 0)
    plsc.subcore_barrier()

    def dblk(k, carry):
        pltpu.sync_copy(acc.at[pl.ds(base + k * CH, CH)], bufa)
        pltpu.sync_copy(bufa, out_h.at[c].at[pl.ds(base + k * CH, CH)])
        return carry

    lax.fori_loop(0, NZB, dblk, 0)


def _sc_deg_body(dst_h, out_h, dst_v, ones_v, zb, acc):
    c = lax.axis_index("c")
    s = lax.axis_index("s")
    w = c * NS + s
    nch = dst_v.shape[0]
    pltpu.sync_copy(dst_h.at[w], dst_v)

    one16 = jnp.ones((16,), jnp.float32)
    for k in range(CH // 16):
        ones_v[pl.ds(k * 16, 16)] = one16
    z16 = jnp.zeros((16,), jnp.float32)
    for k in range(ZR // 16):
        zb[pl.ds(k * 16, 16)] = z16
    base = s * ZR
    pltpu.sync_copy(zb, acc.at[pl.ds(base, ZR)])
    plsc.subcore_barrier()

    def step(j, carry):
        pltpu.sync_copy(ones_v, acc.at[dst_v.at[j]], add=True)
        return carry

    lax.fori_loop(0, nch, step, 0)
    plsc.subcore_barrier()

    pltpu.sync_copy(acc.at[pl.ds(base, ZR)], zb)
    pltpu.sync_copy(zb, out_h.at[c].at[pl.ds(base, ZR)])


@functools.cache
def _mesh():
    return plsc.VectorSubcoreMesh(core_axis_name="c", subcore_axis_name="s",
                                  num_cores=NC, num_subcores=NS)


def _sc_scatter(y, src_g, dst_g):
    nch = src_g.shape[1]
    f = pl.kernel(
        _sc_scatter_body,
        out_type=jax.ShapeDtypeStruct((NC, ACC, D), jnp.float32),
        mesh=_mesh(),
        scratch_types=[
            pltpu.VMEM((nch, CH), jnp.int32),
            pltpu.VMEM((nch, CH), jnp.int32),
            pltpu.VMEM((CH, D), jnp.float32),
            pltpu.VMEM_SHARED((ACC, D), jnp.float32),
            pltpu.SemaphoreType.DMA,
        ],
    )
    return f(y, src_g, dst_g)


def _sc_deg(dst_g):
    nch = dst_g.shape[1]
    f = pl.kernel(
        _sc_deg_body,
        out_type=jax.ShapeDtypeStruct((NC, ACC), jnp.float32),
        mesh=_mesh(),
        scratch_types=[
            pltpu.VMEM((nch, CH), jnp.int32),
            pltpu.VMEM((CH,), jnp.float32),
            pltpu.VMEM((ZR,), jnp.float32),
            pltpu.VMEM_SHARED((ACC,), jnp.float32),
        ],
    )
    return f(dst_g)


def _tc_prep_body(deg_ref, x_ref, w_ref, y_ref, dinv_ref):
    deg = deg_ref[0, :, :] + deg_ref[1, :, :]
    dinv = lax.rsqrt(jnp.maximum(deg, 1.0))
    y_ref[...] = jnp.dot(x_ref[...] * dinv, w_ref[...],
                         preferred_element_type=jnp.float32)
    dinv_ref[...] = dinv


def _tc_mid_body(s_ref, dinv_ref, b_ref, w_ref, y_ref):
    dinv = dinv_ref[...]
    a = (s_ref[0, :, :] + s_ref[1, :, :]) * dinv + b_ref[...]
    mu = jnp.mean(a, axis=1, keepdims=True)
    d = a - mu
    var = jnp.mean(d * d, axis=1, keepdims=True)
    h = d * lax.rsqrt(var + 1e-5)
    h = jnp.maximum(h, 0.0)
    y_ref[...] = jnp.dot(h * dinv, w_ref[...],
                         preferred_element_type=jnp.float32)


def _tc_last_body(s_ref, dinv_ref, b_ref, o_ref):
    a = (s_ref[0, :, :] + s_ref[1, :, :]) * dinv_ref[...] + b_ref[...]
    mu = jnp.mean(a, axis=1, keepdims=True)
    d = a - mu
    var = jnp.mean(d * d, axis=1, keepdims=True)
    o_ref[...] = d * lax.rsqrt(var + 1e-5)


def _tc_prep(deg2, feat, W0):
    return pl.pallas_call(
        _tc_prep_body,
        grid=(N // BLK,),
        in_specs=[
            pl.BlockSpec((2, BLK, 1), lambda i: (0, i, 0)),
            pl.BlockSpec((BLK, D), lambda i: (i, 0)),
            pl.BlockSpec((D, D), lambda i: (0, 0)),
        ],
        out_specs=[
            pl.BlockSpec((BLK, D), lambda i: (i, 0)),
            pl.BlockSpec((BLK, 1), lambda i: (i, 0)),
        ],
        out_shape=[
            jax.ShapeDtypeStruct((N, D), jnp.float32),
            jax.ShapeDtypeStruct((N, 1), jnp.float32),
        ],
    )(deg2, feat, W0)


def _tc_mid(s2, dinv, b, Wn):
    return pl.pallas_call(
        _tc_mid_body,
        grid=(N // BLK,),
        in_specs=[
            pl.BlockSpec((2, BLK, D), lambda i: (0, i, 0)),
            pl.BlockSpec((BLK, 1), lambda i: (i, 0)),
            pl.BlockSpec((1, D), lambda i: (0, 0)),
            pl.BlockSpec((D, D), lambda i: (0, 0)),
        ],
        out_specs=pl.BlockSpec((BLK, D), lambda i: (i, 0)),
        out_shape=jax.ShapeDtypeStruct((N, D), jnp.float32),
    )(s2, dinv, b, Wn)


def _tc_last(s2, dinv, b):
    return pl.pallas_call(
        _tc_last_body,
        grid=(N // BLK,),
        in_specs=[
            pl.BlockSpec((2, BLK, D), lambda i: (0, i, 0)),
            pl.BlockSpec((BLK, 1), lambda i: (i, 0)),
            pl.BlockSpec((1, D), lambda i: (0, 0)),
        ],
        out_specs=pl.BlockSpec((BLK, D), lambda i: (i, 0)),
        out_shape=jax.ShapeDtypeStruct((N, D), jnp.float32),
    )(s2, dinv, b)


def kernel(feat, edge_index, W0, b0, W1, b1, W2, b2):
    feat = feat.astype(jnp.float32)
    src = edge_index[0].astype(jnp.int32)
    dst = edge_index[1].astype(jnp.int32)
    e = src.shape[0]
    gran = NW * CH
    epad = gran * ((e + gran - 1) // gran)
    src_g = jnp.concatenate(
        [src, jnp.zeros((epad - e,), jnp.int32)]).reshape(NW, -1, CH)
    pad_dst = DUMMY + jnp.arange(epad - e, dtype=jnp.int32) % (ACC - N)
    dst_g = jnp.concatenate([dst, pad_dst]).reshape(NW, -1, CH)

    deg2 = _sc_deg(dst_g)
    y, dinv = _tc_prep(deg2.reshape(NC, ACC, 1), feat, W0)
    s2 = _sc_scatter(y, src_g, dst_g)
    y = _tc_mid(s2, dinv, b0.reshape(1, D), W1)
    s2 = _sc_scatter(y, src_g, dst_g)
    y = _tc_mid(s2, dinv, b1.reshape(1, D), W2)
    s2 = _sc_scatter(y, src_g, dst_g)
    return _tc_last(s2, dinv, b2.reshape(1, D))
